# Initial kernel scaffold; baseline (speedup 1.0000x reference)
import jax
import jax.numpy as jnp
from jax.experimental import pallas as pl


def kernel(x, sa1_W1, sa1_D1, sa1_g1, sa1_b1, sa1_W2, sa1_D2, sa1_g2, sa1_b2, sa2_W1, sa2_D1, sa2_g1, sa2_b1, sa2_W2, sa2_D2, sa2_g2, sa2_b2, pred_W):
    raise NotImplementedError("write your pallas kernel here")


# speedup vs baseline: 1.0238x; 1.0238x over previous
import functools

import jax
import jax.numpy as jnp
import numpy as np
from jax.experimental import pallas as pl

EPS = 1e-6
NEG = 0.2


def _lin3(W, X3):
    if W.shape[1] == 1:
        return tuple(W * Xv for Xv in X3)
    return tuple(jnp.dot(W, Xv, preferred_element_type=jnp.float32)
                 for Xv in X3)


def _norm3(X3):
    n2 = X3[0] * X3[0] + X3[1] * X3[1] + X3[2] * X3[2]
    return jnp.sqrt(n2) + EPS


def _bn3(X3, norm, gamma, beta, mean, var):
    nb = gamma * (norm - mean) / jnp.sqrt(var + 1e-5) + beta
    return tuple(Xv / norm * nb for Xv in X3)


def _lrelu3(X3, D):
    D3 = _lin3(D, X3)
    dot = X3[0] * D3[0] + X3[1] * D3[1] + X3[2] * D3[2]
    dsq = D3[0] * D3[0] + D3[1] * D3[1] + D3[2] * D3[2]
    mask = (dot >= 0).astype(jnp.float32)
    out = []
    for Xv, Dv in zip(X3, D3):
        xneg = Xv - (dot / (dsq + EPS)) * Dv
        out.append(NEG * Xv + (1.0 - NEG) * (mask * Xv + (1.0 - mask) * xneg))
    return tuple(out)


def _stats1_kernel(g_ref, W1_ref, out_ref):
    X = tuple(g_ref[0, v] for v in range(3))
    Y = _lin3(W1_ref[...], X)
    n = _norm3(Y)
    out_ref[0, 0, 0, :] = jnp.sum(n, axis=1)
    out_ref[0, 0, 1, :] = jnp.sum(n * n, axis=1)


def _stats2_kernel(g_ref, W1_ref, p1_ref, D1_ref, W2_ref, out_ref):
    X = tuple(g_ref[0, v] for v in range(3))
    Y1 = _lin3(W1_ref[...], X)
    n1 = _norm3(Y1)
    p1 = p1_ref[...]
    X1 = _bn3(Y1, n1, p1[:, 0:1], p1[:, 1:2], p1[:, 2:3], p1[:, 3:4])
    X1 = _lrelu3(X1, D1_ref[...])
    Y2 = _lin3(W2_ref[...], X1)
    n2 = _norm3(Y2)
    out_ref[0, 0, 0, :] = jnp.sum(n2, axis=1)
    out_ref[0, 0, 1, :] = jnp.sum(n2 * n2, axis=1)


def _final_kernel(K, g_ref, W1_ref, p1_ref, D1_ref,
                  W2_ref, p2_ref, D2_ref, out_ref):
    X = tuple(g_ref[0, v] for v in range(3))
    Y1 = _lin3(W1_ref[...], X)
    n1 = _norm3(Y1)
    p1 = p1_ref[...]
    X1 = _bn3(Y1, n1, p1[:, 0:1], p1[:, 1:2], p1[:, 2:3], p1[:, 3:4])
    X1 = _lrelu3(X1, D1_ref[...])
    Y2 = _lin3(W2_ref[...], X1)
    n2 = _norm3(Y2)
    p2 = p2_ref[...]
    X2 = _bn3(Y2, n2, p2[:, 0:1], p2[:, 1:2], p2[:, 2:3], p2[:, 3:4])
    X2 = _lrelu3(X2, D2_ref[...])

    C2, M = X2[0].shape
    S = M // K
    sq = X2[0] * X2[0] + X2[1] * X2[1] + X2[2] * X2[2]
    sqr = sq.reshape(C2, S, K)
    m = jnp.max(sqr, axis=2, keepdims=True)
    ki = jax.lax.broadcasted_iota(jnp.int32, (C2, S, K), 2)
    ksel = jnp.min(jnp.where(sqr >= m, ki, K), axis=2, keepdims=True)
    oh = (ki == ksel).astype(jnp.float32)
    for v in range(3):
        out_ref[0, v] = jnp.sum(X2[v].reshape(C2, S, K) * oh, axis=2)


def _sa_mlp(grouped, W1, D1, g1, b1, W2, D2, g2, b2, S, K, s_blk):
    B = grouped.shape[0]
    C_in = grouped.shape[2]
    C1 = W1.shape[0]
    C2 = W2.shape[0]
    M = S * K
    n_sb = S // s_blk
    m_blk = s_blk * K

    g_spec = pl.BlockSpec((1, 3, C_in, m_blk), lambda b, s: (b, 0, 0, s))
    full = lambda arr: pl.BlockSpec(arr.shape, lambda b, s: (0,) * arr.ndim)
    stat_spec = lambda C: pl.BlockSpec((1, 1, 2, C), lambda b, s: (b, s, 0, 0))

    count = B * S * K

    def coeffs(sums, gamma, beta):
        tot = jnp.sum(sums, axis=(0, 1))
        mean = tot[0] / count
        var = tot[1] / count - mean * mean
        return jnp.stack([gamma, beta, mean, var], axis=1)

    s1 = pl.pallas_call(
        _stats1_kernel,
        grid=(B, n_sb),
        in_specs=[g_spec, full(W1)],
        out_specs=stat_spec(C1),
        out_shape=jax.ShapeDtypeStruct((B, n_sb, 2, C1), jnp.float32),
    )(grouped, W1)
    p1 = coeffs(s1, g1, b1)

    s2 = pl.pallas_call(
        _stats2_kernel,
        grid=(B, n_sb),
        in_specs=[g_spec, full(W1), full(p1), full(D1), full(W2)],
        out_specs=stat_spec(C2),
        out_shape=jax.ShapeDtypeStruct((B, n_sb, 2, C2), jnp.float32),
    )(grouped, W1, p1, D1, W2)
    p2 = coeffs(s2, g2, b2)

    out = pl.pallas_call(
        functools.partial(_final_kernel, K),
        grid=(B, n_sb),
        in_specs=[g_spec, full(W1), full(p1), full(D1),
                  full(W2), full(p2), full(D2)],
        out_specs=pl.BlockSpec((1, 3, C2, s_blk), lambda b, s: (b, 0, 0, s)),
        out_shape=jax.ShapeDtypeStruct((B, 3, C2, S), jnp.float32),
    )(grouped, W1, p1, D1, W2, p2, D2)
    return jnp.transpose(out, (0, 2, 3, 1))


def _fps(xyz, npoint):
    B, N, _ = xyz.shape
    d0 = jnp.full((B, N), 1e10, dtype=xyz.dtype)
    f0 = jnp.zeros((B,), dtype=jnp.int32)

    def step(carry, _):
        dists, far = carry
        c = jnp.take_along_axis(xyz, far[:, None, None], axis=1)
        d = jnp.sum((xyz - c) ** 2, axis=-1)
        dists = jnp.minimum(dists, d)
        nxt = jnp.argmax(dists, axis=-1).astype(jnp.int32)
        return (dists, nxt), far

    _, idxs = jax.lax.scan(step, (d0, f0), None, length=npoint)
    return jnp.transpose(idxs)


def _knn(new_xyz, xyz, k):
    d = (jnp.sum(new_xyz ** 2, -1)[..., None]
         + jnp.sum(xyz ** 2, -1)[:, None, :]
         - 2.0 * jnp.einsum('bsd,bnd->bsn', new_xyz, xyz))
    _, idx = jax.lax.top_k(-d, k)
    return idx


def _gather_feat(feat, idx):
    B, C = feat.shape[0], feat.shape[1]
    S, K = idx.shape[1], idx.shape[2]
    flat = idx.reshape(B, 1, S * K, 1)
    g = jnp.take_along_axis(feat, flat, axis=2)
    return g.reshape(B, C, S, K, 3)


def _l2norm(v):
    n = jnp.linalg.norm(v, axis=-1, keepdims=True)
    return v / jnp.maximum(n, 1e-6)


def kernel(x, sa1_W1, sa1_D1, sa1_g1, sa1_b1, sa1_W2, sa1_D2, sa1_g2, sa1_b2,
           sa2_W1, sa2_D1, sa2_g1, sa2_b1, sa2_W2, sa2_D2, sa2_g2, sa2_b2,
           pred_W):
    B = x.shape[0]

    xyz_t = jnp.transpose(x, (0, 2, 1))
    fi1 = _fps(xyz_t, 256)
    new_xyz1 = jnp.take_along_axis(xyz_t, fi1[..., None], axis=1)
    idx1 = _knn(new_xyz1, xyz_t, 32)
    pts = xyz_t[:, None, :, :]
    grouped1 = _gather_feat(pts, idx1) - new_xyz1[:, None, :, None, :]
    g1_in = jnp.transpose(grouped1, (0, 4, 1, 2, 3)).reshape(B, 3, 1, 256 * 32)
    feat1 = _sa_mlp(g1_in, sa1_W1, sa1_D1, sa1_g1, sa1_b1,
                    sa1_W2, sa1_D2, sa1_g2, sa1_b2, S=256, K=32, s_blk=128)
    xyz1 = jnp.transpose(new_xyz1, (0, 2, 1))

    fi2 = _fps(new_xyz1, 128)
    new_xyz2 = jnp.take_along_axis(new_xyz1, fi2[..., None], axis=1)
    idx2 = _knn(new_xyz2, new_xyz1, 16)
    pts2 = new_xyz1[:, None, :, :]
    ga = _gather_feat(pts2, idx2) - new_xyz2[:, None, :, None, :]
    gb = _gather_feat(feat1, idx2)
    grouped2 = jnp.concatenate([ga, gb], axis=1)
    g2_in = jnp.transpose(grouped2, (0, 4, 1, 2, 3)).reshape(B, 3, 129,
                                                             128 * 16)
    v_feat = _sa_mlp(g2_in, sa2_W1, sa2_D1, sa2_g1, sa2_b1,
                     sa2_W2, sa2_D2, sa2_g2, sa2_b2, S=128, K=16, s_blk=128)
    xyz2 = jnp.transpose(new_xyz2, (0, 2, 1))

    n2 = jnp.sum(v_feat * v_feat, axis=-1)
    pi = jnp.argmax(n2, axis=-1)
    pooled = jnp.take_along_axis(v_feat, pi[..., None, None],
                                 axis=-2)[..., 0, :]
    M = jnp.einsum('oc,bcd->bod', pred_W, pooled)
    v1, v2 = M[:, 0, :], M[:, 1, :]
    u1 = _l2norm(v1)
    u2 = v2 - jnp.sum(u1 * v2, axis=-1, keepdims=True) * u1
    u2 = _l2norm(u2)
    u3 = jnp.cross(u1, u2)
    R_align = jnp.stack([u1, u2, u3], axis=-1)
    return (R_align, v_feat, xyz2, feat1, xyz1)
